# baseline (speedup 1.0000x reference)
import functools

import jax
import jax.numpy as jnp
from jax import lax
from jax.experimental import pallas as pl
from jax.experimental.pallas import tpu as pltpu
from jax.experimental.pallas import tpu_sc as plsc

N_NODES = 10000
N_PAD = 10240
D_FEAT = 128
NC = 2
NS = 16
LANES = 16
K_EDGES = 112
ROWS_PER_SUB = N_PAD // NS
SEGS = [(0, 112), (112, 112), (224, 112), (336, 112), (448, 112), (560, 80)]


def _spmm_sc(rc, wts, dense, c0, c1):
    mesh = plsc.VectorSubcoreMesh(core_axis_name="c", subcore_axis_name="s")

    @functools.partial(
        pl.kernel,
        mesh=mesh,
        out_type=jax.ShapeDtypeStruct((NC, N_PAD, D_FEAT), jnp.float32),
        scratch_types=[
            pltpu.VMEM((3, 2, K_EDGES), jnp.int32),
            pltpu.VMEM((3, K_EDGES), jnp.float32),
            pltpu.VMEM((3, K_EDGES), jnp.int32),
            pltpu.VMEM((3, K_EDGES, D_FEAT), jnp.float32),
            pltpu.VMEM_SHARED((N_PAD, D_FEAT), jnp.float32),
            pltpu.SemaphoreType.DMA,
            pltpu.SemaphoreType.DMA,
            pltpu.SemaphoreType.DMA,
            pltpu.SemaphoreType.DMA,
            pltpu.SemaphoreType.DMA,
            pltpu.SemaphoreType.DMA,
            pltpu.SemaphoreType.DMA,
            pltpu.SemaphoreType.DMA,
            pltpu.SemaphoreType.DMA,
        ],
    )
    def spmm(
        rc_hbm, w_hbm, dense_hbm, out_hbm,
        idx, wbuf, sidx, rows, acc,
        isem0, isem1, isem2, gsem0, gsem1, gsem2, ssem0, ssem1, ssem2,
    ):
        cid = lax.axis_index("c")
        sid = lax.axis_index("s")
        isems = (isem0, isem1, isem2)
        gsems = (gsem0, gsem1, gsem2)
        ssems = (ssem0, ssem1, ssem2)

        cbase = jnp.where(cid == 0, sid * c0, NS * c0 + sid * c1)
        count = jnp.where(cid == 0, c0, c1)

        def start_idx(j, t):
            pltpu.async_copy(rc_hbm.at[cbase + j], idx.at[t], isems[t])
            pltpu.async_copy(w_hbm.at[cbase + j], wbuf.at[t], isems[t])

        def wait_idx(j, t):
            pltpu.make_async_copy(
                rc_hbm.at[cbase + j], idx.at[t], isems[t]
            ).wait()
            pltpu.make_async_copy(
                w_hbm.at[cbase + j], wbuf.at[t], isems[t]
            ).wait()

        def start_gather(t):
            pltpu.async_copy(dense_hbm.at[idx.at[t, 1]], rows.at[t], gsems[t])

        def wait_gather(t):
            pltpu.make_async_copy(
                dense_hbm.at[idx.at[t, 1]], rows.at[t], gsems[t]
            ).wait()

        def wait_scatter(ts):
            pltpu.make_async_copy(
                rows.at[ts], acc.at[sidx.at[ts]], ssems[ts]
            ).wait()

        for t in range(3):
            start_idx(t, t)

        def zero_body(r, _):
            for v in range(D_FEAT // LANES):
                rows[2, r, pl.ds(v * LANES, LANES)] = jnp.zeros(
                    (LANES,), jnp.float32
                )
            return 0

        lax.fori_loop(0, K_EDGES, zero_body, 0)
        for t in range(2):
            wait_idx(t, t)
            start_gather(t)
        for off, sz in SEGS:
            base = sid * ROWS_PER_SUB + off
            pltpu.sync_copy(rows.at[2, pl.ds(0, sz)], acc.at[pl.ds(base, sz)])
        plsc.subcore_barrier()

        def tri_body(trip, _):
            for t in range(3):
                j = trip * 3 + t

                @pl.when(j < count)
                def _():
                    wait_gather(t)

                    def scale_body(g, _c):
                        wv = wbuf[t, pl.ds(g * LANES, LANES)]
                        for l in range(LANES):
                            e = g * LANES + l
                            wvec = jnp.full((LANES,), wv[l], jnp.float32)
                            for v in range(D_FEAT // LANES):
                                sl = pl.ds(v * LANES, LANES)
                                rows[t, e, sl] = rows[t, e, sl] * wvec
                        return 0

                    lax.fori_loop(0, K_EDGES // LANES, scale_body, 0)

                    for g in range(K_EDGES // LANES):
                        sl = pl.ds(g * LANES, LANES)
                        sidx[t, sl] = idx[t, 0, sl]
                    pltpu.async_copy(
                        rows.at[t], acc.at[sidx.at[t]], ssems[t], add=True
                    )

                    @pl.when(j + 3 < count)
                    def _():
                        start_idx(j + 3, t)

                    @pl.when(j + 2 < count)
                    def _():
                        @pl.when(j >= 1)
                        def _():
                            wait_scatter((t + 2) % 3)

                        wait_idx(j + 2, (t + 2) % 3)
                        start_gather((t + 2) % 3)

            return 0

        lax.fori_loop(0, (count + 2) // 3, tri_body, 0)
        wait_scatter(0)
        wait_scatter(1)
        wait_scatter(2)
        plsc.subcore_barrier()

        for off, sz in SEGS:
            base = sid * ROWS_PER_SUB + off
            pltpu.sync_copy(
                acc.at[pl.ds(base, sz)],
                out_hbm.at[cid, pl.ds(base, sz)],
            )

    return spmm(rc, wts, dense)


BN = 1000


def _mm_body(x_ref, w_ref, o_ref):
    o_ref[...] = jnp.dot(x_ref[...], w_ref[...], preferred_element_type=jnp.float32)


def _matmul(x, w):
    n, d = x.shape
    return pl.pallas_call(
        _mm_body,
        grid=(n // BN,),
        in_specs=[
            pl.BlockSpec((BN, d), lambda i: (i, 0)),
            pl.BlockSpec((d, w.shape[1]), lambda i: (0, 0)),
        ],
        out_specs=pl.BlockSpec((BN, w.shape[1]), lambda i: (i, 0)),
        out_shape=jax.ShapeDtypeStruct((n, w.shape[1]), jnp.float32),
    )(x, w)


def _fuse_relu_mm_body(p0_ref, p1_ref, b_ref, w_ref, o_ref):
    h = jax.nn.relu(p0_ref[0] + p1_ref[0] + b_ref[...])
    o_ref[...] = jnp.dot(h, w_ref[...], preferred_element_type=jnp.float32)


def _fuse_relu_mm(p, b, w):
    d = p.shape[2]
    return pl.pallas_call(
        _fuse_relu_mm_body,
        grid=(N_NODES // BN,),
        in_specs=[
            pl.BlockSpec((1, BN, d), lambda i: (0, i, 0)),
            pl.BlockSpec((1, BN, d), lambda i: (1, i, 0)),
            pl.BlockSpec((1, d), lambda i: (0, 0)),
            pl.BlockSpec((d, w.shape[1]), lambda i: (0, 0)),
        ],
        out_specs=pl.BlockSpec((BN, w.shape[1]), lambda i: (i, 0)),
        out_shape=jax.ShapeDtypeStruct((N_NODES, w.shape[1]), jnp.float32),
    )(p, p, b, w)


def _mu_logvar_body(q0_ref, q1_ref, b_ref, mu_ref, lv_ref):
    t = q0_ref[0] + q1_ref[0] + b_ref[...]
    mu_ref[...] = t[:, : D_FEAT // 2]
    lv_ref[...] = t[:, D_FEAT // 2 :]


def _mu_logvar(q, bc):
    d = q.shape[2]
    h = d // 2
    return pl.pallas_call(
        _mu_logvar_body,
        grid=(N_NODES // BN,),
        in_specs=[
            pl.BlockSpec((1, BN, d), lambda i: (0, i, 0)),
            pl.BlockSpec((1, BN, d), lambda i: (1, i, 0)),
            pl.BlockSpec((1, d), lambda i: (0, 0)),
        ],
        out_specs=[
            pl.BlockSpec((BN, h), lambda i: (i, 0)),
            pl.BlockSpec((BN, h), lambda i: (i, 0)),
        ],
        out_shape=[
            jax.ShapeDtypeStruct((N_NODES, h), jnp.float32),
            jax.ShapeDtypeStruct((N_NODES, h), jnp.float32),
        ],
    )(q, q, bc)


def _gram_body(a_ref, b_ref, o_ref):
    o_ref[...] = lax.dot_general(
        a_ref[...],
        b_ref[...],
        (((1,), (1,)), ((), ())),
        preferred_element_type=jnp.float32,
    )


BM_GRAM = 80


def _gram(z):
    n, d = z.shape
    return pl.pallas_call(
        _gram_body,
        grid=(n // BM_GRAM,),
        in_specs=[
            pl.BlockSpec((BM_GRAM, d), lambda i: (i, 0)),
            pl.BlockSpec((n, d), lambda i: (0, 0)),
        ],
        out_specs=pl.BlockSpec((BM_GRAM, n), lambda i: (i, 0)),
        out_shape=jax.ShapeDtypeStruct((n, n), jnp.float32),
    )(z, z)


def kernel(x, edge_index, edge_weight, W1, b1, W2, b2, W3, b3):
    row = edge_index[0].astype(jnp.int32)
    col = edge_index[1].astype(jnp.int32)
    w = edge_weight.astype(jnp.float32)

    e = row.shape[0]
    tot = -(-e // (NS * K_EDGES))
    c0 = (tot * 52 + 50) // 100
    c1 = tot - c0
    e_pad = NS * K_EDGES * tot
    pad = e_pad - e
    if pad:
        row = jnp.concatenate([row, jnp.zeros((pad,), jnp.int32)])
        col = jnp.concatenate([col, jnp.zeros((pad,), jnp.int32)])
        w = jnp.concatenate([w, jnp.zeros((pad,), jnp.float32)])
    rc = jnp.stack(
        [
            row.reshape(NS * tot, K_EDGES),
            col.reshape(NS * tot, K_EDGES),
        ],
        axis=1,
    )
    wts = w.reshape(NS * tot, K_EDGES)

    support1 = _matmul(x, W1)
    p = _spmm_sc(rc, wts, support1, c0, c1)
    wc = jnp.concatenate([W2, W3], axis=1)
    bc1 = b1.reshape(1, D_FEAT)
    hw = _fuse_relu_mm(p, bc1, wc)
    q = _spmm_sc(rc, wts, hw, c0, c1)
    bc23 = jnp.concatenate([b2, b3]).reshape(1, D_FEAT)
    mu, logvar = _mu_logvar(q, bc23)
    recon = _gram(mu)
    return (recon, mu, logvar, mu)


# speedup vs baseline: 1.0270x; 1.0270x over previous
import functools

import jax
import jax.numpy as jnp
from jax import lax
from jax.experimental import pallas as pl
from jax.experimental.pallas import tpu as pltpu
from jax.experimental.pallas import tpu_sc as plsc

N_NODES = 10000
N_PAD = 10240
D_FEAT = 128
NC = 2
NS = 16
LANES = 16
K_EDGES = 112
ROWS_PER_SUB = N_PAD // NS
SEGS = [(0, 112), (112, 112), (224, 112), (336, 112), (448, 112), (560, 80)]


def _spmm_sc(rc, wts, dense, c0, c1):
    mesh = plsc.VectorSubcoreMesh(core_axis_name="c", subcore_axis_name="s")

    @functools.partial(
        pl.kernel,
        mesh=mesh,
        out_type=jax.ShapeDtypeStruct((NC, N_PAD, D_FEAT), jnp.float32),
        scratch_types=[
            pltpu.VMEM((3, 2, K_EDGES), jnp.int32),
            pltpu.VMEM((3, K_EDGES), jnp.float32),
            pltpu.VMEM((3, K_EDGES), jnp.int32),
            pltpu.VMEM((3, K_EDGES, D_FEAT), jnp.float32),
            pltpu.VMEM_SHARED((N_PAD, D_FEAT), jnp.float32),
            pltpu.SemaphoreType.DMA,
            pltpu.SemaphoreType.DMA,
            pltpu.SemaphoreType.DMA,
            pltpu.SemaphoreType.DMA,
            pltpu.SemaphoreType.DMA,
            pltpu.SemaphoreType.DMA,
            pltpu.SemaphoreType.DMA,
            pltpu.SemaphoreType.DMA,
            pltpu.SemaphoreType.DMA,
        ],
    )
    def spmm(
        rc_hbm, w_hbm, dense_hbm, out_hbm,
        idx, wbuf, sidx, rows, acc,
        isem0, isem1, isem2, gsem0, gsem1, gsem2, ssem0, ssem1, ssem2,
    ):
        cid = lax.axis_index("c")
        sid = lax.axis_index("s")
        isems = (isem0, isem1, isem2)
        gsems = (gsem0, gsem1, gsem2)
        ssems = (ssem0, ssem1, ssem2)

        cbase = jnp.where(cid == 0, sid * c0, NS * c0 + sid * c1)
        count = jnp.where(cid == 0, c0, c1)

        def start_idx(j, t):
            pltpu.async_copy(rc_hbm.at[cbase + j], idx.at[t], isems[t])
            pltpu.async_copy(w_hbm.at[cbase + j], wbuf.at[t], isems[t])

        def wait_idx(j, t):
            pltpu.make_async_copy(
                rc_hbm.at[cbase + j], idx.at[t], isems[t]
            ).wait()
            pltpu.make_async_copy(
                w_hbm.at[cbase + j], wbuf.at[t], isems[t]
            ).wait()

        def start_gather(t):
            pltpu.async_copy(dense_hbm.at[idx.at[t, 1]], rows.at[t], gsems[t])

        def wait_gather(t):
            pltpu.make_async_copy(
                dense_hbm.at[idx.at[t, 1]], rows.at[t], gsems[t]
            ).wait()

        def wait_scatter(ts):
            pltpu.make_async_copy(
                rows.at[ts], acc.at[sidx.at[ts]], ssems[ts]
            ).wait()

        for t in range(3):
            start_idx(t, t)

        def zero_body(r, _):
            for v in range(D_FEAT // LANES):
                rows[2, r, pl.ds(v * LANES, LANES)] = jnp.zeros(
                    (LANES,), jnp.float32
                )
            return 0

        lax.fori_loop(0, K_EDGES, zero_body, 0)
        for t in range(2):
            wait_idx(t, t)
            start_gather(t)
        for off, sz in SEGS:
            base = sid * ROWS_PER_SUB + off
            pltpu.sync_copy(rows.at[2, pl.ds(0, sz)], acc.at[pl.ds(base, sz)])
        plsc.subcore_barrier()

        def tri_body(trip, _):
            for t in range(3):
                j = trip * 3 + t

                @pl.when(j < count)
                def _():
                    wait_gather(t)

                    def scale_body(g, _c):
                        wv = wbuf[t, pl.ds(g * LANES, LANES)]
                        for l in range(LANES):
                            e = g * LANES + l
                            wvec = jnp.full((LANES,), wv[l], jnp.float32)
                            for v in range(D_FEAT // LANES):
                                sl = pl.ds(v * LANES, LANES)
                                rows[t, e, sl] = rows[t, e, sl] * wvec
                        return 0

                    lax.fori_loop(0, K_EDGES // LANES, scale_body, 0)

                    for g in range(K_EDGES // LANES):
                        sl = pl.ds(g * LANES, LANES)
                        sidx[t, sl] = idx[t, 0, sl]
                    pltpu.async_copy(
                        rows.at[t], acc.at[sidx.at[t]], ssems[t], add=True
                    )

                    @pl.when(j + 3 < count)
                    def _():
                        start_idx(j + 3, t)

                    @pl.when(j + 2 < count)
                    def _():
                        @pl.when(j >= 1)
                        def _():
                            wait_scatter((t + 2) % 3)

                        wait_idx(j + 2, (t + 2) % 3)
                        start_gather((t + 2) % 3)

            return 0

        lax.fori_loop(0, (count + 2) // 3, tri_body, 0)
        wait_scatter(0)
        wait_scatter(1)
        wait_scatter(2)
        plsc.subcore_barrier()

        for off, sz in SEGS:
            base = sid * ROWS_PER_SUB + off
            pltpu.sync_copy(
                acc.at[pl.ds(base, sz)],
                out_hbm.at[cid, pl.ds(base, sz)],
            )

    return spmm(rc, wts, dense)


BN = 1000


def _mm_body(x_ref, w_ref, o_ref):
    o_ref[...] = jnp.dot(x_ref[...], w_ref[...], preferred_element_type=jnp.float32)


def _matmul(x, w):
    n, d = x.shape
    return pl.pallas_call(
        _mm_body,
        grid=(n // BN,),
        in_specs=[
            pl.BlockSpec((BN, d), lambda i: (i, 0)),
            pl.BlockSpec((d, w.shape[1]), lambda i: (0, 0)),
        ],
        out_specs=pl.BlockSpec((BN, w.shape[1]), lambda i: (i, 0)),
        out_shape=jax.ShapeDtypeStruct((n, w.shape[1]), jnp.float32),
    )(x, w)


def _fuse_relu_mm_body(p0_ref, p1_ref, b_ref, w_ref, o_ref):
    h = jax.nn.relu(p0_ref[0] + p1_ref[0] + b_ref[...])
    o_ref[...] = jnp.dot(h, w_ref[...], preferred_element_type=jnp.float32)


def _fuse_relu_mm(p, b, w):
    d = p.shape[2]
    return pl.pallas_call(
        _fuse_relu_mm_body,
        grid=(N_NODES // BN,),
        in_specs=[
            pl.BlockSpec((1, BN, d), lambda i: (0, i, 0)),
            pl.BlockSpec((1, BN, d), lambda i: (1, i, 0)),
            pl.BlockSpec((1, d), lambda i: (0, 0)),
            pl.BlockSpec((d, w.shape[1]), lambda i: (0, 0)),
        ],
        out_specs=pl.BlockSpec((BN, w.shape[1]), lambda i: (i, 0)),
        out_shape=jax.ShapeDtypeStruct((N_NODES, w.shape[1]), jnp.float32),
    )(p, p, b, w)


def _mu_logvar_body(q0_ref, q1_ref, b_ref, mu_ref, lv_ref):
    t = q0_ref[0] + q1_ref[0] + b_ref[...]
    mu_ref[...] = t[:, : D_FEAT // 2]
    lv_ref[...] = t[:, D_FEAT // 2 :]


def _mu_logvar(q, bc):
    d = q.shape[2]
    h = d // 2
    return pl.pallas_call(
        _mu_logvar_body,
        grid=(N_NODES // BN,),
        in_specs=[
            pl.BlockSpec((1, BN, d), lambda i: (0, i, 0)),
            pl.BlockSpec((1, BN, d), lambda i: (1, i, 0)),
            pl.BlockSpec((1, d), lambda i: (0, 0)),
        ],
        out_specs=[
            pl.BlockSpec((BN, h), lambda i: (i, 0)),
            pl.BlockSpec((BN, h), lambda i: (i, 0)),
        ],
        out_shape=[
            jax.ShapeDtypeStruct((N_NODES, h), jnp.float32),
            jax.ShapeDtypeStruct((N_NODES, h), jnp.float32),
        ],
    )(q, q, bc)


def _gram_body(a_ref, b_ref, o_ref):
    o_ref[...] = lax.dot_general(
        a_ref[...],
        b_ref[...],
        (((1,), (1,)), ((), ())),
        preferred_element_type=jnp.float32,
    )


BM_GRAM = 80


def _gram(z):
    n, d = z.shape
    return pl.pallas_call(
        _gram_body,
        grid=(n // BM_GRAM,),
        in_specs=[
            pl.BlockSpec((BM_GRAM, d), lambda i: (i, 0)),
            pl.BlockSpec((n, d), lambda i: (0, 0)),
        ],
        out_specs=pl.BlockSpec((BM_GRAM, n), lambda i: (i, 0)),
        out_shape=jax.ShapeDtypeStruct((n, n), jnp.float32),
    )(z, z)


def kernel(x, edge_index, edge_weight, W1, b1, W2, b2, W3, b3):
    row = edge_index[0].astype(jnp.int32)
    col = edge_index[1].astype(jnp.int32)
    w = edge_weight.astype(jnp.float32)

    e = row.shape[0]
    tot = -(-e // (NS * K_EDGES))
    c0 = (tot * 55 + 50) // 100
    c1 = tot - c0
    e_pad = NS * K_EDGES * tot
    pad = e_pad - e
    if pad:
        row = jnp.concatenate([row, jnp.zeros((pad,), jnp.int32)])
        col = jnp.concatenate([col, jnp.zeros((pad,), jnp.int32)])
        w = jnp.concatenate([w, jnp.zeros((pad,), jnp.float32)])
    rc = jnp.stack(
        [
            row.reshape(NS * tot, K_EDGES),
            col.reshape(NS * tot, K_EDGES),
        ],
        axis=1,
    )
    wts = w.reshape(NS * tot, K_EDGES)

    support1 = _matmul(x, W1)
    p = _spmm_sc(rc, wts, support1, c0, c1)
    wc = jnp.concatenate([W2, W3], axis=1)
    bc1 = b1.reshape(1, D_FEAT)
    hw = _fuse_relu_mm(p, bc1, wc)
    q = _spmm_sc(rc, wts, hw, c0, c1)
    bc23 = jnp.concatenate([b2, b3]).reshape(1, D_FEAT)
    mu, logvar = _mu_logvar(q, bc23)
    recon = _gram(mu)
    return (recon, mu, logvar, mu)
